# baseline (speedup 1.0000x reference)
import functools

import jax
import jax.numpy as jnp
from jax import lax
from jax.experimental import pallas as pl
from jax.experimental.pallas import tpu as pltpu
from jax.experimental.pallas import tpu_sc as plsc

N = 10000
E = 320000
D = 128
U = 1000
NC = 2
NS = 16
NW = NC * NS
EPW = E // NW
K1 = 100
NCH1 = EPW // K1
NP = 10240
K = 128
EPP = 10240
NCH = EPP // K
G = 8
NG = NCH // G
N2 = 1024

_mesh = plsc.VectorSubcoreMesh(core_axis_name="c", subcore_axis_name="s")


@functools.partial(
    pl.kernel,
    out_type=jax.ShapeDtypeStruct((NC, N), jnp.float32),
    mesh=_mesh,
    scratch_types=[
        pltpu.VMEM((NCH1, K1), jnp.int32),
        pltpu.VMEM((112,), jnp.float32),
        pltpu.VMEM((2000,), jnp.float32),
        pltpu.VMEM_SHARED((N,), jnp.float32),
    ],
)
def _deg_kernel(dst_hbm, deg_hbm, idx_v, ones_v, zb_v, acc_sh):
    cid = lax.axis_index("c")
    sid = lax.axis_index("s")
    pltpu.sync_copy(dst_hbm.at[cid, sid], idx_v)

    for i in range(112 // 16):
        ones_v[pl.ds(i * 16, 16)] = jnp.ones((16,), jnp.float32)

    @pl.when(sid == 0)
    def _():
        z = jnp.zeros((16,), jnp.float32)

        @pl.loop(0, 125)
        def _(r):
            zb_v[pl.ds(r * 16, 16)] = z

        for i in range(5):
            pltpu.sync_copy(zb_v, acc_sh.at[pl.ds(i * 2000, 2000)])

    plsc.subcore_barrier()

    @pl.loop(0, NCH1)
    def _(j):
        pltpu.sync_copy(ones_v.at[pl.ds(0, K1)], acc_sh.at[idx_v.at[j]],
                        add=True)

    plsc.subcore_barrier()

    @pl.when(sid == 0)
    def _():
        pltpu.sync_copy(acc_sh, deg_hbm.at[cid])


@functools.partial(
    pl.kernel,
    out_type=jax.ShapeDtypeStruct((NC, NP, D), jnp.float32),
    mesh=_mesh,
    scratch_types=[
        pltpu.VMEM((NCH // 2, K), jnp.int32),
        pltpu.VMEM((NCH // 2, K), jnp.int32),
        pltpu.VMEM((2, K, D), jnp.float32),
        pltpu.VMEM_SHARED((NP, D), jnp.float32),
        pltpu.SemaphoreType.DMA,
        pltpu.SemaphoreType.DMA,
    ],
)
def _edge_kernel(y_hbm, src_hbm, dst_hbm, out_hbm, sidx_v, didx_v, buf_v,
                 acc_sh, sem0, sem1):
    cid = lax.axis_index("c")
    sid = lax.axis_index("s")
    sems = (sem0, sem1)
    HC = NCH // 2

    z = jnp.zeros((16,), jnp.float32)

    @pl.loop(0, 16)
    def _(r):
        for c in range(D // 16):
            buf_v[0, r, pl.ds(c * 16, 16)] = z

    @pl.loop(sid, NP // 16, step=NS)
    def _(i):
        pltpu.sync_copy(buf_v.at[0, pl.ds(0, 16)], acc_sh.at[pl.ds(i * 16, 16)])

    plsc.subcore_barrier()

    def issue(j, b):
        pltpu.async_copy(y_hbm.at[sidx_v.at[j]], buf_v.at[b], sems[b])

    def wait_data(b):
        pltpu.make_async_copy(y_hbm.at[sidx_v.at[0]], buf_v.at[b],
                              sems[b]).wait()

    for half in range(2):
        pltpu.sync_copy(src_hbm.at[cid, sid, pl.ds(half * HC, HC)], sidx_v)
        pltpu.sync_copy(dst_hbm.at[cid, sid, pl.ds(half * HC, HC)], didx_v)
        @pl.loop(0, HC)
        def _(j):
            pltpu.async_copy(y_hbm.at[sidx_v.at[j]], buf_v.at[0],
                             sem0).wait()
            pltpu.sync_copy(buf_v.at[0], acc_sh.at[didx_v.at[j]], add=True)

    plsc.subcore_barrier()

    @pl.loop(sid, NP // 16, step=NS)
    def _(i):
        pltpu.sync_copy(acc_sh.at[pl.ds(i * 16, 16)],
                        out_hbm.at[cid, pl.ds(i * 16, 16)])


def _y1_body(x_ref, w_ref, d0_ref, d1_ref, y_ref):
    dis = lax.rsqrt(d0_ref[...] + d1_ref[...] + 1.0)
    xw = jnp.dot(x_ref[...], w_ref[...], preferred_element_type=jnp.float32)
    y_ref[...] = xw * dis


def _mid_body(p0_ref, p1_ref, y_ref, d0_ref, d1_ref, b_ref, w_ref, out_ref):
    dis = lax.rsqrt(d0_ref[...] + d1_ref[...] + 1.0)
    t = dis * (p0_ref[...] + p1_ref[...] + y_ref[...]) + b_ref[...]
    h = jnp.where(t > 0, t, jnp.exp(t) - 1.0)
    hw = jnp.dot(h, w_ref[...], preferred_element_type=jnp.float32)
    out_ref[...] = hw * dis


def _final_body(q0_ref, q1_ref, y_ref, d0_ref, d1_ref, b_ref, out_ref):
    dis = lax.rsqrt(d0_ref[...] + d1_ref[...] + 1.0)
    t = dis * (q0_ref[...] + q1_ref[...] + y_ref[...]) + b_ref[...]
    out_ref[...] = jnp.where(t > 0, t, jnp.exp(t) - 1.0)


def kernel(x, edge_index, W1, b1, W2, b2):
    ei = edge_index.astype(jnp.int32)
    src = ei[0]
    dst = ei[1]
    dst1 = dst.reshape(NC, NS, NCH1, K1)

    pad = jnp.full((NW, EPP - EPW), N, jnp.int32)
    srcp = jnp.concatenate([src.reshape(NW, EPW), pad], axis=1)
    srcp = srcp.reshape(NC, NS, NCH, K)
    dstp = jnp.concatenate([dst.reshape(NW, EPW), pad], axis=1)
    dstp = dstp.reshape(NC, NS, NCH, K)

    deg_p = _deg_kernel(dst1)
    dpad = jnp.zeros((NC, NP - N), jnp.float32)
    degp = jnp.concatenate([deg_p, dpad], axis=1)
    d0 = degp[0].reshape(NP, 1)
    d1 = degp[1].reshape(NP, 1)

    xp = jnp.concatenate([x, jnp.zeros((NP - N, D), x.dtype)], axis=0)

    R = 320
    grid = NP // R
    row_spec = pl.BlockSpec((R, D), lambda i: (i, 0))
    dcol_spec = pl.BlockSpec((R, 1), lambda i: (i, 0))
    full_spec = pl.BlockSpec((D, D), lambda i: (0, 0))
    bias_spec = pl.BlockSpec((1, D), lambda i: (0, 0))

    y1 = pl.pallas_call(
        _y1_body,
        grid=(grid,),
        in_specs=[row_spec, full_spec, dcol_spec, dcol_spec],
        out_specs=row_spec,
        out_shape=jax.ShapeDtypeStruct((NP, D), jnp.float32),
    )(xp, W1, d0, d1)

    p = _edge_kernel(y1, srcp, dstp)

    y2 = pl.pallas_call(
        _mid_body,
        grid=(grid,),
        in_specs=[row_spec, row_spec, row_spec, dcol_spec, dcol_spec,
                  bias_spec, full_spec],
        out_specs=row_spec,
        out_shape=jax.ShapeDtypeStruct((NP, D), jnp.float32),
    )(p[0], p[1], y1, d0, d1, b1.reshape(1, D), W2)

    q = _edge_kernel(y2, srcp, dstp)

    RT = 128
    top_spec = pl.BlockSpec((RT, D), lambda i: (i, 0))
    top_dcol = pl.BlockSpec((RT, 1), lambda i: (i, 0))
    top_bias = pl.BlockSpec((1, D), lambda i: (0, 0))
    out_top = pl.pallas_call(
        _final_body,
        grid=(N2 // RT,),
        in_specs=[top_spec, top_spec, top_spec, top_dcol, top_dcol, top_bias],
        out_specs=top_spec,
        out_shape=jax.ShapeDtypeStruct((N2, D), jnp.float32),
    )(q[0, :N2], q[1, :N2], y2[:N2], d0[:N2], d1[:N2], b2.reshape(1, D))

    return jnp.concatenate([out_top[:U], x[U:]], axis=0)


# speedup vs baseline: 2.7968x; 2.7968x over previous
import functools

import jax
import jax.numpy as jnp
from jax import lax
from jax.experimental import pallas as pl
from jax.experimental.pallas import tpu as pltpu
from jax.experimental.pallas import tpu_sc as plsc

N = 10000
E = 320000
D = 128
U = 1000
NC = 2
NS = 16
NW = NC * NS
EPW = E // NW
K1 = 100
NCH1 = EPW // K1
NP = 10240
K = 100
EPP = 10000
NCH = EPP // K
G = 8
NG = NCH // G
N2 = 1024

_mesh = plsc.VectorSubcoreMesh(core_axis_name="c", subcore_axis_name="s")


@functools.partial(
    pl.kernel,
    out_type=jax.ShapeDtypeStruct((NC, N), jnp.float32),
    mesh=_mesh,
    scratch_types=[
        pltpu.VMEM((NCH1, K1), jnp.int32),
        pltpu.VMEM((112,), jnp.float32),
        pltpu.VMEM((2000,), jnp.float32),
        pltpu.VMEM_SHARED((N,), jnp.float32),
    ],
)
def _deg_kernel(dst_hbm, deg_hbm, idx_v, ones_v, zb_v, acc_sh):
    cid = lax.axis_index("c")
    sid = lax.axis_index("s")
    pltpu.sync_copy(dst_hbm.at[cid, sid], idx_v)

    for i in range(112 // 16):
        ones_v[pl.ds(i * 16, 16)] = jnp.ones((16,), jnp.float32)

    @pl.when(sid == 0)
    def _():
        z = jnp.zeros((16,), jnp.float32)

        @pl.loop(0, 125)
        def _(r):
            zb_v[pl.ds(r * 16, 16)] = z

        for i in range(5):
            pltpu.sync_copy(zb_v, acc_sh.at[pl.ds(i * 2000, 2000)])

    plsc.subcore_barrier()

    @pl.loop(0, NCH1)
    def _(j):
        pltpu.sync_copy(ones_v.at[pl.ds(0, K1)], acc_sh.at[idx_v.at[j]],
                        add=True)

    plsc.subcore_barrier()

    @pl.when(sid == 0)
    def _():
        pltpu.sync_copy(acc_sh, deg_hbm.at[cid])


@functools.partial(
    pl.kernel,
    out_type=jax.ShapeDtypeStruct((NC, NP, D), jnp.float32),
    mesh=_mesh,
    scratch_types=[
        pltpu.VMEM((52, K), jnp.int32),
        pltpu.VMEM((52, K), jnp.int32),
        pltpu.VMEM((2, K, D), jnp.float32),
        pltpu.VMEM_SHARED((NP, D), jnp.float32),
        pltpu.SemaphoreType.DMA,
        pltpu.SemaphoreType.DMA,
    ],
)
def _edge_kernel(y_hbm, src_hbm, dst_hbm, out_hbm, sidx_v, didx_v, buf_v,
                 acc_sh, sem0, sem1):
    cid = lax.axis_index("c")
    sid = lax.axis_index("s")
    sems = (sem0, sem1)

    z = jnp.zeros((16,), jnp.float32)

    @pl.loop(0, 16)
    def _(r):
        for c in range(D // 16):
            buf_v[0, r, pl.ds(c * 16, 16)] = z

    @pl.loop(sid, NP // 16, step=NS)
    def _(i):
        pltpu.sync_copy(buf_v.at[0, pl.ds(0, 16)], acc_sh.at[pl.ds(i * 16, 16)])

    plsc.subcore_barrier()

    def issue(j, b):
        pltpu.async_copy(y_hbm.at[sidx_v.at[j]], buf_v.at[b], sems[b])

    def wait_data(b):
        pltpu.make_async_copy(y_hbm.at[sidx_v.at[0]], buf_v.at[b],
                              sems[b]).wait()

    for off, hc in ((0, 48), (48, 52)):
        pltpu.sync_copy(src_hbm.at[cid, sid, pl.ds(off, hc)],
                        sidx_v.at[pl.ds(0, hc)])
        pltpu.sync_copy(dst_hbm.at[cid, sid, pl.ds(off, hc)],
                        didx_v.at[pl.ds(0, hc)])
        issue(0, 0)
        issue(1, 1)

        @pl.loop(0, hc - 2, step=2)
        def _(j):
            for b in range(2):
                wait_data(b)
                pltpu.sync_copy(buf_v.at[b], acc_sh.at[didx_v.at[j + b]],
                                add=True)
                issue(j + b + 2, b)

        for b in range(2):
            wait_data(b)
            pltpu.sync_copy(buf_v.at[b], acc_sh.at[didx_v.at[hc - 2 + b]],
                            add=True)

    plsc.subcore_barrier()

    @pl.loop(sid, NP // 16, step=NS)
    def _(i):
        pltpu.sync_copy(acc_sh.at[pl.ds(i * 16, 16)],
                        out_hbm.at[cid, pl.ds(i * 16, 16)])


def _y1_body(x_ref, w_ref, d0_ref, d1_ref, y_ref):
    dis = lax.rsqrt(d0_ref[...] + d1_ref[...] + 1.0)
    xw = jnp.dot(x_ref[...], w_ref[...], preferred_element_type=jnp.float32)
    y_ref[...] = xw * dis


def _mid_body(p0_ref, p1_ref, y_ref, d0_ref, d1_ref, b_ref, w_ref, out_ref):
    dis = lax.rsqrt(d0_ref[...] + d1_ref[...] + 1.0)
    t = dis * (p0_ref[...] + p1_ref[...] + y_ref[...]) + b_ref[...]
    h = jnp.where(t > 0, t, jnp.exp(t) - 1.0)
    hw = jnp.dot(h, w_ref[...], preferred_element_type=jnp.float32)
    out_ref[...] = hw * dis


def _final_body(q0_ref, q1_ref, y_ref, d0_ref, d1_ref, b_ref, out_ref):
    dis = lax.rsqrt(d0_ref[...] + d1_ref[...] + 1.0)
    t = dis * (q0_ref[...] + q1_ref[...] + y_ref[...]) + b_ref[...]
    out_ref[...] = jnp.where(t > 0, t, jnp.exp(t) - 1.0)


def kernel(x, edge_index, W1, b1, W2, b2):
    ei = edge_index.astype(jnp.int32)
    src = ei[0]
    dst = ei[1]
    dst1 = dst.reshape(NC, NS, NCH1, K1)

    pad = jnp.full((NW, EPP - EPW), N, jnp.int32)
    srcp = jnp.concatenate([src.reshape(NW, EPW), pad], axis=1)
    srcp = srcp.reshape(NC, NS, NCH, K)
    dstp = jnp.concatenate([dst.reshape(NW, EPW), pad], axis=1)
    dstp = dstp.reshape(NC, NS, NCH, K)

    deg_p = _deg_kernel(dst1)
    dpad = jnp.zeros((NC, NP - N), jnp.float32)
    degp = jnp.concatenate([deg_p, dpad], axis=1)
    d0 = degp[0].reshape(NP, 1)
    d1 = degp[1].reshape(NP, 1)

    xp = jnp.concatenate([x, jnp.zeros((NP - N, D), x.dtype)], axis=0)

    R = 320
    grid = NP // R
    row_spec = pl.BlockSpec((R, D), lambda i: (i, 0))
    dcol_spec = pl.BlockSpec((R, 1), lambda i: (i, 0))
    full_spec = pl.BlockSpec((D, D), lambda i: (0, 0))
    bias_spec = pl.BlockSpec((1, D), lambda i: (0, 0))

    y1 = pl.pallas_call(
        _y1_body,
        grid=(grid,),
        in_specs=[row_spec, full_spec, dcol_spec, dcol_spec],
        out_specs=row_spec,
        out_shape=jax.ShapeDtypeStruct((NP, D), jnp.float32),
    )(xp, W1, d0, d1)

    p = _edge_kernel(y1, srcp, dstp)

    y2 = pl.pallas_call(
        _mid_body,
        grid=(grid,),
        in_specs=[row_spec, row_spec, row_spec, dcol_spec, dcol_spec,
                  bias_spec, full_spec],
        out_specs=row_spec,
        out_shape=jax.ShapeDtypeStruct((NP, D), jnp.float32),
    )(p[0], p[1], y1, d0, d1, b1.reshape(1, D), W2)

    q = _edge_kernel(y2, srcp, dstp)

    RT = 128
    top_spec = pl.BlockSpec((RT, D), lambda i: (i, 0))
    top_dcol = pl.BlockSpec((RT, 1), lambda i: (i, 0))
    top_bias = pl.BlockSpec((1, D), lambda i: (0, 0))
    out_top = pl.pallas_call(
        _final_body,
        grid=(N2 // RT,),
        in_specs=[top_spec, top_spec, top_spec, top_dcol, top_dcol, top_bias],
        out_specs=top_spec,
        out_shape=jax.ShapeDtypeStruct((N2, D), jnp.float32),
    )(q[0, :N2], q[1, :N2], y2[:N2], d0[:N2], d1[:N2], b2.reshape(1, D))

    return jnp.concatenate([out_top[:U], x[U:]], axis=0)
